# Initial kernel scaffold; baseline (speedup 1.0000x reference)
import jax
import jax.numpy as jnp
from jax.experimental import pallas as pl


def kernel(coords, mask, residue_pdb_idx):
    raise NotImplementedError("write your pallas kernel here")


# speedup vs baseline: 21.0701x; 21.0701x over previous
import functools

import jax
import jax.numpy as jnp
import numpy as np
from jax.experimental import pallas as pl

_LIMS_F32 = np.linspace(-np.float32(np.pi), np.float32(np.pi), 20).astype(np.float32)
_COS_THRESH = [float(np.float32(np.cos(np.float64(_LIMS_F32[k])))) for k in range(10, 19)]


def _body(x_ref, idx_ref, o_ref, *, n):
    x = x_ref[0]

    def col(j):
        return x[:, j : j + 1]

    nx, ny, nz = col(0), col(1), col(2)
    cax, cay, caz = col(3), col(4), col(5)
    cx, cy, cz = col(6), col(7), col(8)

    def nxt(a):
        return jnp.concatenate([a[1:], a[:1]], axis=0)

    def cos_angle(ax, ay, az, bx, by, bz, gx, gy, gz):
        v1x, v1y, v1z = ax - bx, ay - by, az - bz
        v2x, v2y, v2z = gx - bx, gy - by, gz - bz
        dot = v1x * v2x + v1y * v2y + v1z * v2z
        s1 = v1x * v1x + v1y * v1y + v1z * v1z
        s2 = v2x * v2x + v2y * v2y + v2z * v2z
        return dot / (jnp.sqrt(s1) * jnp.sqrt(s2) + 1e-10)

    nxn, nyn, nzn = nxt(nx), nxt(ny), nxt(nz)
    caxn, cayn, cazn = nxt(cax), nxt(cay), nxt(caz)

    cos1 = cos_angle(nx, ny, nz, cax, cay, caz, cx, cy, cz)
    cos2 = cos_angle(cax, cay, caz, cx, cy, cz, nxn, nyn, nzn)
    cos3 = cos_angle(cx, cy, cz, nxn, nyn, nzn, caxn, cayn, cazn)

    idx = idx_ref[0]
    row = jax.lax.broadcasted_iota(jnp.int32, (n, 1), 0)
    good = ((nxt(idx) - idx) == 1) & (row < (n - 1))
    cos2 = jnp.where(good, cos2, 2.0)
    cos3 = jnp.where(good, cos3, 2.0)

    def bin_of(c):
        acc = jnp.full((n, 1), 10, dtype=jnp.int32)
        for t in _COS_THRESH:
            acc = acc + (c < t).astype(jnp.int32)
        return acc

    b1 = bin_of(cos1)
    b2 = bin_of(cos2) + 21
    b3 = bin_of(cos3) + 42

    cols = jax.lax.broadcasted_iota(jnp.int32, (n, 63), 1)
    onehot = (cols == b1) | (cols == b2) | (cols == b3)
    o_ref[0] = onehot.astype(jnp.float32)


def kernel(coords, mask, residue_pdb_idx):
    del mask
    b, n = coords.shape[0], coords.shape[1]
    x = coords.reshape(b, n, coords.shape[2] * coords.shape[3])
    idx = residue_pdb_idx.astype(jnp.int32)[..., None]

    return pl.pallas_call(
        functools.partial(_body, n=n),
        grid=(b,),
        in_specs=[
            pl.BlockSpec((1, n, x.shape[2]), lambda i: (i, 0, 0)),
            pl.BlockSpec((1, n, 1), lambda i: (i, 0, 0)),
        ],
        out_specs=pl.BlockSpec((1, n, 63), lambda i: (i, 0, 0)),
        out_shape=jax.ShapeDtypeStruct((b, n, 63), jnp.float32),
    )(x, idx)


# speedup vs baseline: 100.5522x; 4.7723x over previous
import functools

import jax
import jax.numpy as jnp
import numpy as np
from jax.experimental import pallas as pl

_LIMS_F32 = np.linspace(-np.float32(np.pi), np.float32(np.pi), 20).astype(np.float32)
_COS_THRESH = [float(np.float32(np.cos(np.float64(_LIMS_F32[k])))) for k in range(10, 19)]


def _body(x_ref, idx_ref, o_ref, *, n, s):
    def comp(j):
        return x_ref[0, j]

    def nxtflat(a):
        col0 = a[:, 0:1]
        col0s = jnp.concatenate([col0[1:], col0[:1]], axis=0)
        return jnp.concatenate([a[:, 1:], col0s], axis=1)

    nx, ny, nz = comp(0), comp(1), comp(2)
    cax, cay, caz = comp(3), comp(4), comp(5)
    cx, cy, cz = comp(6), comp(7), comp(8)
    nxn, nyn, nzn = nxtflat(nx), nxtflat(ny), nxtflat(nz)
    caxn, cayn, cazn = nxtflat(cax), nxtflat(cay), nxtflat(caz)

    def cos_angle(ax, ay, az, bx, by, bz, gx, gy, gz):
        v1x, v1y, v1z = ax - bx, ay - by, az - bz
        v2x, v2y, v2z = gx - bx, gy - by, gz - bz
        dot = v1x * v2x + v1y * v2y + v1z * v2z
        s1 = v1x * v1x + v1y * v1y + v1z * v1z
        s2 = v2x * v2x + v2y * v2y + v2z * v2z
        return dot / (jnp.sqrt(s1) * jnp.sqrt(s2) + 1e-10)

    cos1 = cos_angle(nx, ny, nz, cax, cay, caz, cx, cy, cz)
    cos2 = cos_angle(cax, cay, caz, cx, cy, cz, nxn, nyn, nzn)
    cos3 = cos_angle(cx, cy, cz, nxn, nyn, nzn, caxn, cayn, cazn)

    idx = idx_ref[0, 0]
    sub = jax.lax.broadcasted_iota(jnp.int32, (s, 128), 0)
    lane = jax.lax.broadcasted_iota(jnp.int32, (s, 128), 1)
    good = ((nxtflat(idx) - idx) == 1) & ~((sub == s - 1) & (lane == 127))
    cos2 = jnp.where(good, cos2, 2.0)
    cos3 = jnp.where(good, cos3, 2.0)

    def bin_of(c):
        acc = jnp.full((s, 128), 10, dtype=jnp.int32)
        for t in _COS_THRESH:
            acc = acc + (c < t).astype(jnp.int32)
        return acc

    b1, b2, b3 = bin_of(cos1), bin_of(cos2), bin_of(cos3)

    zero = jnp.zeros((s, 128), dtype=jnp.float32)
    for c in range(63):
        if 10 <= c <= 19:
            v = (b1 == c).astype(jnp.float32)
        elif 31 <= c <= 40:
            v = (b2 == c - 21).astype(jnp.float32)
        elif 52 <= c <= 61:
            v = (b3 == c - 42).astype(jnp.float32)
        else:
            v = zero
        o_ref[0, c] = v


def kernel(coords, mask, residue_pdb_idx):
    del mask
    b, n = coords.shape[0], coords.shape[1]
    assert n % 128 == 0
    s = n // 128
    nca = coords[:, :, :3, :].reshape(b, n, 9)
    xt = nca.transpose(0, 2, 1).reshape(b, 9, s, 128)
    idxt = residue_pdb_idx.astype(jnp.int32).reshape(b, 1, s, 128)

    ot = pl.pallas_call(
        functools.partial(_body, n=n, s=s),
        grid=(b,),
        in_specs=[
            pl.BlockSpec((1, 9, s, 128), lambda i: (i, 0, 0, 0)),
            pl.BlockSpec((1, 1, s, 128), lambda i: (i, 0, 0, 0)),
        ],
        out_specs=pl.BlockSpec((1, 63, s, 128), lambda i: (i, 0, 0, 0)),
        out_shape=jax.ShapeDtypeStruct((b, 63, s, 128), jnp.float32),
    )(xt, idxt)
    return ot.reshape(b, 63, n).transpose(0, 2, 1)
